# baseline (speedup 1.0000x reference)
import functools

import jax
import jax.numpy as jnp
from jax import lax
from jax.experimental import pallas as pl

D_MODEL = 768
NUM_EXPERTS = 8
TOP_K = 2

_BR = 1024


def _router_body(x_ref, w_ref, probs_ref, idx_ref):
    xb = x_ref[...]
    w = w_ref[...]
    scores = lax.dot_general(
        xb, w, (((1,), (1,)), ((), ())),
        preferred_element_type=jnp.float32)

    e_idx = lax.broadcasted_iota(jnp.int32, scores.shape, 1)
    m1 = jnp.max(scores, axis=1, keepdims=True)
    i1 = jnp.min(jnp.where(scores == m1, e_idx, NUM_EXPERTS),
                 axis=1, keepdims=True)
    neg = jnp.float32(-jnp.inf)
    masked = jnp.where(e_idx == i1, neg, scores)
    m2 = jnp.max(masked, axis=1, keepdims=True)
    i2 = jnp.min(jnp.where(masked == m2, e_idx, NUM_EXPERTS),
                 axis=1, keepdims=True)
    t = jnp.exp(m2 - m1)
    denom = 1.0 + t
    p1 = 1.0 / denom
    p2 = t / denom
    probs_ref[...] = jnp.concatenate([p1, p2], axis=1)
    idx_ref[...] = jnp.concatenate([i1, i2], axis=1)


def kernel(x, W):
    b, s, d = x.shape
    n = b * s
    x_flat = x.reshape(n, d)
    grid = n // _BR
    probs, idx = pl.pallas_call(
        _router_body,
        grid=(grid,),
        in_specs=[
            pl.BlockSpec((_BR, d), lambda i: (i, 0)),
            pl.BlockSpec((NUM_EXPERTS, d), lambda i: (0, 0)),
        ],
        out_specs=[
            pl.BlockSpec((_BR, TOP_K), lambda i: (i, 0)),
            pl.BlockSpec((_BR, TOP_K), lambda i: (i, 0)),
        ],
        out_shape=[
            jax.ShapeDtypeStruct((n, TOP_K), jnp.float32),
            jax.ShapeDtypeStruct((n, TOP_K), jnp.int32),
        ],
    )(x_flat, W)
    return probs, idx
